# baseline (speedup 1.0000x reference)
import functools

import jax
import jax.numpy as jnp
from jax import lax
from jax.experimental import pallas as pl
from jax.experimental.pallas import tpu as pltpu
from jax.experimental.pallas import tpu_sc as plsc

N = 10000
E = 320000
D = 128
NC = 2
NS = 16
CH = 128
N_PAD = 10112
ROWS_PER_TILE = N_PAD // NS
E_PER_TILE = 160 * CH
E_PAD = NS * E_PER_TILE
DEG_PER_TILE = E // NS
DEG_STAGE = 2000


def _sc_deg_body(dst_hbm, out_hbm, hist_v, stage_v):
    c = lax.axis_index("c")
    s = lax.axis_index("s")

    @pl.loop(0, N_PAD // 16)
    def _zero(i):
        hist_v[pl.ds(i * 16, 16)] = jnp.zeros((16,), jnp.float32)

    ones = jnp.ones((16,), jnp.float32)
    base = c * E + s * DEG_PER_TILE

    @pl.loop(0, DEG_PER_TILE // DEG_STAGE)
    def _outer(k):
        pltpu.sync_copy(dst_hbm.at[pl.ds(base + k * DEG_STAGE, DEG_STAGE)],
                        stage_v)

        @pl.loop(0, DEG_STAGE // 16)
        def _inner(j):
            idx = stage_v[pl.ds(j * 16, 16)]
            plsc.addupdate_scatter(hist_v, [idx], ones)

    pltpu.sync_copy(hist_v, out_hbm.at[pl.ds((c * NS + s) * N_PAD, N_PAD)])


def _sc_deg(dst_flat):
    mesh = plsc.VectorSubcoreMesh(core_axis_name="c", subcore_axis_name="s")
    fn = pl.kernel(
        _sc_deg_body,
        out_type=jax.ShapeDtypeStruct((NC * NS * N_PAD,), jnp.float32),
        mesh=mesh,
        scratch_types=[
            pltpu.VMEM((N_PAD,), jnp.float32),
            pltpu.VMEM((DEG_STAGE,), jnp.int32),
        ],
        compiler_params=pltpu.CompilerParams(needs_layout_passes=False),
    )
    return fn(dst_flat)


def _tc_matmul_body(x0_ref, x1_ref, w0_ref, w1_ref, h_ref):
    h_ref[0] = jnp.dot(x0_ref[...], w0_ref[...],
                       preferred_element_type=jnp.float32)
    h_ref[1] = jnp.dot(x1_ref[...], w1_ref[...],
                       preferred_element_type=jnp.float32)


def _tc_matmul(x0, x1, w0, w1):
    bn = 2000
    grid = (N // bn,)
    return pl.pallas_call(
        _tc_matmul_body,
        grid=grid,
        in_specs=[
            pl.BlockSpec((bn, D), lambda i: (i, 0)),
            pl.BlockSpec((bn, D), lambda i: (i, 0)),
            pl.BlockSpec((D, D), lambda i: (0, 0)),
            pl.BlockSpec((D, D), lambda i: (0, 0)),
        ],
        out_specs=pl.BlockSpec((NC, bn, D), lambda i: (0, i, 0)),
        out_shape=jax.ShapeDtypeStruct((NC, N, D), jnp.float32),
    )(x0, x1, w0, w1)


def _tc_prescale_body(h_ref, degp_ref, g_ref, dinv_ref):
    dinv0 = lax.rsqrt(jnp.sum(degp_ref[0], axis=0) + 1.0)
    dinv1 = lax.rsqrt(jnp.sum(degp_ref[1], axis=0) + 1.0)
    dinv_ref[:, 0:1] = dinv0.reshape(-1, 1)
    dinv_ref[:, 1:2] = dinv1.reshape(-1, 1)
    g_ref[0] = h_ref[0] * dinv0[:N].reshape(-1, 1)
    g_ref[1] = h_ref[1] * dinv1[:N].reshape(-1, 1)


def _tc_prescale(h, degp):
    return pl.pallas_call(
        _tc_prescale_body,
        out_shape=[
            jax.ShapeDtypeStruct((NC, N, D), jnp.float32),
            jax.ShapeDtypeStruct((N_PAD, NC), jnp.float32),
        ],
    )(h, degp)


CHUNKS = E_PER_TILE // CH


def _sc_agg_body(g_hbm, idxp_hbm, zeros_hbm, out_hbm,
                 acc_sh, ibuf_v, rows_v, sem_i, sem_g, sem_s0, sem_s1):
    c = lax.axis_index("c")
    s = lax.axis_index("s")
    w = c * NS + s

    r0 = s * ROWS_PER_TILE
    pltpu.sync_copy(zeros_hbm.at[pl.ds(r0, ROWS_PER_TILE)],
                    acc_sh.at[pl.ds(r0, ROWS_PER_TILE)])

    crow = w * CHUNKS
    sem_s = (sem_s0, sem_s1)

    pltpu.sync_copy(idxp_hbm.at[crow], ibuf_v.at[0])
    pltpu.async_copy(idxp_hbm.at[crow + 1], ibuf_v.at[1], sem_i)
    pltpu.async_copy(idxp_hbm.at[crow + 2], ibuf_v.at[2], sem_i)
    plsc.subcore_barrier()
    pltpu.async_copy(g_hbm.at[ibuf_v.at[0, 0]], rows_v.at[0], sem_g)

    @pl.loop(0, CHUNKS, step=4)
    def _chunk(i0):
        for q in range(4):
            i = i0 + q
            b = q & 1
            b2 = 1 - b
            q1 = (q + 1) % 4
            q3 = (q + 3) % 4
            @pl.when(i + 1 < CHUNKS)
            def _():
                pltpu.make_async_copy(
                    idxp_hbm.at[crow], ibuf_v.at[q1], sem_i).wait()
                @pl.when(i >= 1)
                def _():
                    pltpu.make_async_copy(
                        g_hbm.at[pl.ds(0, CH)], rows_v.at[b2],
                        sem_s[b2]).wait()
                @pl.when(i + 3 < CHUNKS)
                def _():
                    pltpu.async_copy(
                        idxp_hbm.at[crow + i + 3], ibuf_v.at[q3], sem_i)
                pltpu.async_copy(
                    g_hbm.at[ibuf_v.at[q1, 0]], rows_v.at[b2], sem_g)

            pltpu.make_async_copy(
                g_hbm.at[ibuf_v.at[q, 0]], rows_v.at[b], sem_g).wait()
            pltpu.async_copy(rows_v.at[b], acc_sh.at[ibuf_v.at[q, 1]],
                             sem_s[b], add=True)

    pltpu.make_async_copy(g_hbm.at[pl.ds(0, CH)], rows_v.at[0], sem_s0).wait()
    pltpu.make_async_copy(g_hbm.at[pl.ds(0, CH)], rows_v.at[1], sem_s1).wait()

    plsc.subcore_barrier()
    pltpu.sync_copy(acc_sh.at[pl.ds(r0, ROWS_PER_TILE)],
                    out_hbm.at[pl.ds(c * N_PAD + r0, ROWS_PER_TILE)])


def _sc_agg(g_flat, idx_pair, zeros_init):
    mesh = plsc.VectorSubcoreMesh(core_axis_name="c", subcore_axis_name="s")
    fn = pl.kernel(
        _sc_agg_body,
        out_type=jax.ShapeDtypeStruct((NC * N_PAD, D), jnp.float32),
        mesh=mesh,
        scratch_types=[
            pltpu.VMEM_SHARED((N_PAD, D), jnp.float32),
            pltpu.VMEM((4, 2, CH), jnp.int32),
            pltpu.VMEM((2, CH, D), jnp.float32),
            pltpu.SemaphoreType.DMA,
            pltpu.SemaphoreType.DMA,
            pltpu.SemaphoreType.DMA,
            pltpu.SemaphoreType.DMA,
        ],
    )
    return fn(g_flat, idx_pair, zeros_init)


def _tc_finish_body(agg_ref, g_ref, dinv_ref, b0_ref, b1_ref,
                    wfc1_ref, bfc1_ref, wfc2_ref, bfc2_ref,
                    out_ref, acc_s):
    i = pl.program_id(0)

    @pl.when(i == 0)
    def _():
        acc_s[...] = jnp.zeros_like(acc_s)

    a0 = agg_ref[0].astype(jnp.float32) + g_ref[0].astype(jnp.float32)
    a1 = agg_ref[1].astype(jnp.float32) + g_ref[1].astype(jnp.float32)
    o0 = a0 * dinv_ref[:, 0:1] + b0_ref[...]
    o1 = a1 * dinv_ref[:, 1:2] + b1_ref[...]
    o0 = jnp.maximum(o0, 0.0)
    o1 = jnp.maximum(o1, 0.0)
    acc_s[0:1, :] += jnp.sum(o0, axis=0).reshape(1, D)
    acc_s[1:2, :] += jnp.sum(o1, axis=0).reshape(1, D)

    @pl.when(i == pl.num_programs(0) - 1)
    def _():
        hm = acc_s[0:1, :] * (1.0 / N)
        hi = acc_s[1:2, :] * (1.0 / N)
        h = jnp.dot(hm, wfc1_ref[0:D, :], preferred_element_type=jnp.float32)
        h += jnp.dot(hi, wfc1_ref[D:2 * D, :],
                     preferred_element_type=jnp.float32)
        h = jnp.maximum(h + bfc1_ref[...], 0.0)
        o = jnp.dot(h, wfc2_ref[...], preferred_element_type=jnp.float32)
        o = o + bfc2_ref[...]
        out_ref[...] = jax.nn.sigmoid(o)


def _tc_finish(agg, g, dinv, b0, b1, wfc1, bfc1, wfc2, bfc2):
    bn = 2000
    grid = (N // bn,)
    return pl.pallas_call(
        _tc_finish_body,
        grid=grid,
        in_specs=[
            pl.BlockSpec((NC, bn, D), lambda i: (0, i, 0)),
            pl.BlockSpec((NC, bn, D), lambda i: (0, i, 0)),
            pl.BlockSpec((bn, NC), lambda i: (i, 0)),
            pl.BlockSpec((1, D), lambda i: (0, 0)),
            pl.BlockSpec((1, D), lambda i: (0, 0)),
            pl.BlockSpec((2 * D, D), lambda i: (0, 0)),
            pl.BlockSpec((1, D), lambda i: (0, 0)),
            pl.BlockSpec((D, 1), lambda i: (0, 0)),
            pl.BlockSpec((1, 1), lambda i: (0, 0)),
        ],
        out_specs=pl.BlockSpec((1, 1), lambda i: (0, 0)),
        out_shape=jax.ShapeDtypeStruct((1, 1), jnp.float32),
        scratch_shapes=[pltpu.VMEM((NC, D), jnp.float32)],
    )(agg, g, dinv, b0, b1, wfc1, bfc1, wfc2, bfc2)


@jax.jit
def kernel(x_molecular, edge_index_molecular, x_interaction,
           edge_index_interaction, W_mol, b_mol, W_int, b_int,
           W_fc1, b_fc1, W_fc2, b_fc2):
    ei0 = edge_index_molecular.astype(jnp.int32)
    ei1 = edge_index_interaction.astype(jnp.int32)

    pad = jnp.zeros((E_PAD - E,), jnp.int32)
    padn = jnp.full((E_PAD - E,), N, jnp.int32)
    src_flat = jnp.concatenate([ei0[0], pad, ei1[0] + N, pad + N])
    dst_flat = jnp.concatenate([ei0[1], padn, ei1[1], padn])
    deg_dst = jnp.concatenate([ei0[1], ei1[1]])
    idx_pair = jnp.stack(
        [src_flat.reshape(-1, CH), dst_flat.reshape(-1, CH)], axis=1)

    h = _tc_matmul(x_molecular, x_interaction, W_mol, W_int)
    degp = _sc_deg(deg_dst).reshape(NC, NS, N_PAD)

    g, dinv = _tc_prescale(h, degp)
    g_flat = g.reshape(NC * N, D)

    zeros_init = jnp.zeros((N_PAD, D), jnp.float32)
    agg = _sc_agg(g_flat, idx_pair, zeros_init)

    out = _tc_finish(agg.reshape(NC, N_PAD, D), g, dinv,
                     b_mol.reshape(1, D), b_int.reshape(1, D),
                     W_fc1, b_fc1.reshape(1, D),
                     W_fc2, b_fc2.reshape(1, 1))
    return out.reshape(1)


# speedup vs baseline: 1.8391x; 1.8391x over previous
import functools

import jax
import jax.numpy as jnp
from jax import lax
from jax.experimental import pallas as pl
from jax.experimental.pallas import tpu as pltpu
from jax.experimental.pallas import tpu_sc as plsc

N = 10000
E = 320000
D = 128
NC = 2
NS = 16
CH = 120
N_PAD = 10112
ROWS_PER_TILE = N_PAD // NS
E_PER_TILE = 168 * CH
E_PAD = NS * E_PER_TILE
DEG_PER_TILE = E // NS
DEG_STAGE = 2000


def _sc_deg_body(dst_hbm, out_hbm, hist_v, stage_v):
    c = lax.axis_index("c")
    s = lax.axis_index("s")

    @pl.loop(0, N_PAD // 16)
    def _zero(i):
        hist_v[pl.ds(i * 16, 16)] = jnp.zeros((16,), jnp.float32)

    ones = jnp.ones((16,), jnp.float32)
    base = c * E + s * DEG_PER_TILE

    @pl.loop(0, DEG_PER_TILE // DEG_STAGE)
    def _outer(k):
        pltpu.sync_copy(dst_hbm.at[pl.ds(base + k * DEG_STAGE, DEG_STAGE)],
                        stage_v)

        @pl.loop(0, DEG_STAGE // 16)
        def _inner(j):
            idx = stage_v[pl.ds(j * 16, 16)]
            plsc.addupdate_scatter(hist_v, [idx], ones)

    pltpu.sync_copy(hist_v, out_hbm.at[pl.ds((c * NS + s) * N_PAD, N_PAD)])


def _sc_deg(dst_flat):
    mesh = plsc.VectorSubcoreMesh(core_axis_name="c", subcore_axis_name="s")
    fn = pl.kernel(
        _sc_deg_body,
        out_type=jax.ShapeDtypeStruct((NC * NS * N_PAD,), jnp.float32),
        mesh=mesh,
        scratch_types=[
            pltpu.VMEM((N_PAD,), jnp.float32),
            pltpu.VMEM((DEG_STAGE,), jnp.int32),
        ],
        compiler_params=pltpu.CompilerParams(needs_layout_passes=False),
    )
    return fn(dst_flat)


def _tc_dinv_body(degp_ref, dinv_ref):
    dinv_ref[:, 0:1] = lax.rsqrt(
        jnp.sum(degp_ref[0], axis=0) + 1.0).reshape(-1, 1)
    dinv_ref[:, 1:2] = lax.rsqrt(
        jnp.sum(degp_ref[1], axis=0) + 1.0).reshape(-1, 1)


def _tc_dinv(degp):
    return pl.pallas_call(
        _tc_dinv_body,
        out_shape=jax.ShapeDtypeStruct((N_PAD, NC), jnp.float32),
    )(degp)


def _tc_prescale_body(x0_ref, x1_ref, w0_ref, w1_ref, dinv_ref, g_ref):
    m0 = jnp.dot(x0_ref[...], w0_ref[...], preferred_element_type=jnp.float32)
    m1 = jnp.dot(x1_ref[...], w1_ref[...], preferred_element_type=jnp.float32)
    g_ref[0] = m0 * dinv_ref[:, 0:1]
    g_ref[1] = m1 * dinv_ref[:, 1:2]


def _tc_prescale(x0, x1, w0, w1, dinv):
    bn = 2000
    grid = (N // bn,)
    return pl.pallas_call(
        _tc_prescale_body,
        grid=grid,
        in_specs=[
            pl.BlockSpec((bn, D), lambda i: (i, 0)),
            pl.BlockSpec((bn, D), lambda i: (i, 0)),
            pl.BlockSpec((D, D), lambda i: (0, 0)),
            pl.BlockSpec((D, D), lambda i: (0, 0)),
            pl.BlockSpec((bn, NC), lambda i: (i, 0)),
        ],
        out_specs=pl.BlockSpec((NC, bn, D), lambda i: (0, i, 0)),
        out_shape=jax.ShapeDtypeStruct((NC, N, D), jnp.float32),
    )(x0, x1, w0, w1, dinv)


CHUNKS = E_PER_TILE // CH


def _sc_agg_body(g_hbm, idxp_hbm, zeros_hbm, out_hbm,
                 acc_sh, ibuf_v, rows_v, sem_i, sem_g,
                 sem_s0, sem_s1, sem_s2):
    c = lax.axis_index("c")
    s = lax.axis_index("s")
    w = c * NS + s

    r0 = s * ROWS_PER_TILE
    pltpu.sync_copy(zeros_hbm.at[pl.ds(r0, ROWS_PER_TILE)],
                    acc_sh.at[pl.ds(r0, ROWS_PER_TILE)])

    crow = w * CHUNKS
    sem_s = (sem_s0, sem_s1, sem_s2)

    pltpu.sync_copy(idxp_hbm.at[crow], ibuf_v.at[0])
    pltpu.async_copy(idxp_hbm.at[crow + 1], ibuf_v.at[1], sem_i)
    pltpu.async_copy(idxp_hbm.at[crow + 2], ibuf_v.at[2], sem_i)
    plsc.subcore_barrier()
    pltpu.async_copy(g_hbm.at[ibuf_v.at[0, 0]], rows_v.at[0], sem_g)
    pltpu.make_async_copy(idxp_hbm.at[crow], ibuf_v.at[1], sem_i).wait()
    pltpu.async_copy(g_hbm.at[ibuf_v.at[1, 0]], rows_v.at[1], sem_g)

    @pl.loop(0, CHUNKS, step=12)
    def _chunk(i0):
        for u in range(12):
            i = i0 + u
            r = u % 3
            q = u % 4
            r2 = (u + 2) % 3
            q2 = (u + 2) % 4
            q3 = (u + 3) % 4

            @pl.when(i + 2 < CHUNKS)
            def _():
                pltpu.make_async_copy(
                    idxp_hbm.at[crow], ibuf_v.at[q2], sem_i).wait()
                @pl.when(i >= 1)
                def _():
                    pltpu.make_async_copy(
                        g_hbm.at[pl.ds(0, CH)], rows_v.at[r2],
                        sem_s[r2]).wait()
                @pl.when(i + 3 < CHUNKS)
                def _():
                    pltpu.async_copy(
                        idxp_hbm.at[crow + i + 3], ibuf_v.at[q3], sem_i)
                pltpu.async_copy(
                    g_hbm.at[ibuf_v.at[q2, 0]], rows_v.at[r2], sem_g)

            pltpu.make_async_copy(
                g_hbm.at[ibuf_v.at[q, 0]], rows_v.at[r], sem_g).wait()
            pltpu.async_copy(rows_v.at[r], acc_sh.at[ibuf_v.at[q, 1]],
                             sem_s[r], add=True)

    pltpu.make_async_copy(g_hbm.at[pl.ds(0, CH)], rows_v.at[0], sem_s0).wait()
    pltpu.make_async_copy(g_hbm.at[pl.ds(0, CH)], rows_v.at[1], sem_s1).wait()
    pltpu.make_async_copy(g_hbm.at[pl.ds(0, CH)], rows_v.at[2], sem_s2).wait()

    plsc.subcore_barrier()
    pltpu.sync_copy(acc_sh.at[pl.ds(r0, ROWS_PER_TILE)],
                    out_hbm.at[pl.ds(c * N_PAD + r0, ROWS_PER_TILE)])


def _sc_agg(g_flat, idx_pair, zeros_init):
    mesh = plsc.VectorSubcoreMesh(core_axis_name="c", subcore_axis_name="s")
    fn = pl.kernel(
        _sc_agg_body,
        out_type=jax.ShapeDtypeStruct((NC * N_PAD, D), jnp.float32),
        mesh=mesh,
        scratch_types=[
            pltpu.VMEM_SHARED((N_PAD, D), jnp.float32),
            pltpu.VMEM((4, 2, CH), jnp.int32),
            pltpu.VMEM((3, CH, D), jnp.float32),
            pltpu.SemaphoreType.DMA,
            pltpu.SemaphoreType.DMA,
            pltpu.SemaphoreType.DMA,
            pltpu.SemaphoreType.DMA,
            pltpu.SemaphoreType.DMA,
        ],
    )
    return fn(g_flat, idx_pair, zeros_init)


def _tc_finish_body(agg_ref, g_ref, dinv_ref, b0_ref, b1_ref,
                    wfc1_ref, bfc1_ref, wfc2_ref, bfc2_ref,
                    out_ref, acc_s):
    i = pl.program_id(0)

    @pl.when(i == 0)
    def _():
        acc_s[...] = jnp.zeros_like(acc_s)

    a0 = agg_ref[0].astype(jnp.float32) + g_ref[0].astype(jnp.float32)
    a1 = agg_ref[1].astype(jnp.float32) + g_ref[1].astype(jnp.float32)
    o0 = a0 * dinv_ref[:, 0:1] + b0_ref[...]
    o1 = a1 * dinv_ref[:, 1:2] + b1_ref[...]
    o0 = jnp.maximum(o0, 0.0)
    o1 = jnp.maximum(o1, 0.0)
    acc_s[0:1, :] += jnp.sum(o0, axis=0).reshape(1, D)
    acc_s[1:2, :] += jnp.sum(o1, axis=0).reshape(1, D)

    @pl.when(i == pl.num_programs(0) - 1)
    def _():
        hm = acc_s[0:1, :] * (1.0 / N)
        hi = acc_s[1:2, :] * (1.0 / N)
        h = jnp.dot(hm, wfc1_ref[0:D, :], preferred_element_type=jnp.float32)
        h += jnp.dot(hi, wfc1_ref[D:2 * D, :],
                     preferred_element_type=jnp.float32)
        h = jnp.maximum(h + bfc1_ref[...], 0.0)
        o = jnp.dot(h, wfc2_ref[...], preferred_element_type=jnp.float32)
        o = o + bfc2_ref[...]
        out_ref[...] = jax.nn.sigmoid(o)


def _tc_finish(agg, g, dinv, b0, b1, wfc1, bfc1, wfc2, bfc2):
    bn = 2000
    grid = (N // bn,)
    return pl.pallas_call(
        _tc_finish_body,
        grid=grid,
        in_specs=[
            pl.BlockSpec((NC, bn, D), lambda i: (0, i, 0)),
            pl.BlockSpec((NC, bn, D), lambda i: (0, i, 0)),
            pl.BlockSpec((bn, NC), lambda i: (i, 0)),
            pl.BlockSpec((1, D), lambda i: (0, 0)),
            pl.BlockSpec((1, D), lambda i: (0, 0)),
            pl.BlockSpec((2 * D, D), lambda i: (0, 0)),
            pl.BlockSpec((1, D), lambda i: (0, 0)),
            pl.BlockSpec((D, 1), lambda i: (0, 0)),
            pl.BlockSpec((1, 1), lambda i: (0, 0)),
        ],
        out_specs=pl.BlockSpec((1, 1), lambda i: (0, 0)),
        out_shape=jax.ShapeDtypeStruct((1, 1), jnp.float32),
        scratch_shapes=[pltpu.VMEM((NC, D), jnp.float32)],
    )(agg, g, dinv, b0, b1, wfc1, bfc1, wfc2, bfc2)


@jax.jit
def kernel(x_molecular, edge_index_molecular, x_interaction,
           edge_index_interaction, W_mol, b_mol, W_int, b_int,
           W_fc1, b_fc1, W_fc2, b_fc2):
    ei0 = edge_index_molecular.astype(jnp.int32)
    ei1 = edge_index_interaction.astype(jnp.int32)

    pad = jnp.zeros((E_PAD - E,), jnp.int32)
    padn = jnp.full((E_PAD - E,), N, jnp.int32)
    src_flat = jnp.concatenate([ei0[0], pad, ei1[0] + N, pad + N])
    dst_flat = jnp.concatenate([ei0[1], padn, ei1[1], padn])
    deg_dst = jnp.concatenate([ei0[1], ei1[1]])
    idx_pair = jnp.stack(
        [src_flat.reshape(-1, CH), dst_flat.reshape(-1, CH)], axis=1)

    degp = _sc_deg(deg_dst).reshape(NC, NS, N_PAD)
    dinv = _tc_dinv(degp)

    g = _tc_prescale(x_molecular, x_interaction, W_mol, W_int, dinv)
    g_flat = g.reshape(NC * N, D)

    zeros_init = jnp.zeros((N_PAD, D), jnp.float32)
    agg = _sc_agg(g_flat, idx_pair, zeros_init)

    out = _tc_finish(agg.reshape(NC, N_PAD, D), g, dinv,
                     b_mol.reshape(1, D), b_int.reshape(1, D),
                     W_fc1, b_fc1.reshape(1, D),
                     W_fc2, b_fc2.reshape(1, 1))
    return out.reshape(1)


# speedup vs baseline: 2.2491x; 1.2230x over previous
import functools

import jax
import jax.numpy as jnp
from jax import lax
from jax.experimental import pallas as pl
from jax.experimental.pallas import tpu as pltpu
from jax.experimental.pallas import tpu_sc as plsc

N = 10000
E = 320000
D = 128
NC = 2
NS = 16
CH = 88
N_PAD = 10112
ROWS_PER_TILE = N_PAD // NS
E_PER_TILE = 228 * CH
E_PAD = NS * E_PER_TILE
DEG_PER_TILE = E // NS
DEG_STAGE = 2000


def _sc_deg_body(dst_hbm, out_hbm, hist_v, stage_v):
    c = lax.axis_index("c")
    s = lax.axis_index("s")

    @pl.loop(0, N_PAD // 16)
    def _zero(i):
        hist_v[pl.ds(i * 16, 16)] = jnp.zeros((16,), jnp.float32)

    ones = jnp.ones((16,), jnp.float32)
    base = c * E + s * DEG_PER_TILE

    @pl.loop(0, DEG_PER_TILE // DEG_STAGE)
    def _outer(k):
        pltpu.sync_copy(dst_hbm.at[pl.ds(base + k * DEG_STAGE, DEG_STAGE)],
                        stage_v)

        @pl.loop(0, DEG_STAGE // 16)
        def _inner(j):
            idx = stage_v[pl.ds(j * 16, 16)]
            plsc.addupdate_scatter(hist_v, [idx], ones)

    pltpu.sync_copy(hist_v, out_hbm.at[pl.ds((c * NS + s) * N_PAD, N_PAD)])


def _sc_deg(dst_flat):
    mesh = plsc.VectorSubcoreMesh(core_axis_name="c", subcore_axis_name="s")
    fn = pl.kernel(
        _sc_deg_body,
        out_type=jax.ShapeDtypeStruct((NC * NS * N_PAD,), jnp.float32),
        mesh=mesh,
        scratch_types=[
            pltpu.VMEM((N_PAD,), jnp.float32),
            pltpu.VMEM((DEG_STAGE,), jnp.int32),
        ],
        compiler_params=pltpu.CompilerParams(needs_layout_passes=False),
    )
    return fn(dst_flat)


def _tc_dinv_body(degp_ref, dinv_ref):
    dinv_ref[:, 0:1] = lax.rsqrt(
        jnp.sum(degp_ref[0], axis=0) + 1.0).reshape(-1, 1)
    dinv_ref[:, 1:2] = lax.rsqrt(
        jnp.sum(degp_ref[1], axis=0) + 1.0).reshape(-1, 1)


def _tc_dinv(degp):
    return pl.pallas_call(
        _tc_dinv_body,
        out_shape=jax.ShapeDtypeStruct((N_PAD, NC), jnp.float32),
    )(degp)


def _tc_prescale_body(x0_ref, x1_ref, w0_ref, w1_ref, dinv_ref, g_ref):
    m0 = jnp.dot(x0_ref[...], w0_ref[...], preferred_element_type=jnp.float32)
    m1 = jnp.dot(x1_ref[...], w1_ref[...], preferred_element_type=jnp.float32)
    g_ref[0] = m0 * dinv_ref[:, 0:1]
    g_ref[1] = m1 * dinv_ref[:, 1:2]


def _tc_prescale(x0, x1, w0, w1, dinv):
    bn = 2000
    grid = (N // bn,)
    return pl.pallas_call(
        _tc_prescale_body,
        grid=grid,
        in_specs=[
            pl.BlockSpec((bn, D), lambda i: (i, 0)),
            pl.BlockSpec((bn, D), lambda i: (i, 0)),
            pl.BlockSpec((D, D), lambda i: (0, 0)),
            pl.BlockSpec((D, D), lambda i: (0, 0)),
            pl.BlockSpec((bn, NC), lambda i: (i, 0)),
        ],
        out_specs=pl.BlockSpec((NC, bn, D), lambda i: (0, i, 0)),
        out_shape=jax.ShapeDtypeStruct((NC, N, D), jnp.float32),
    )(x0, x1, w0, w1, dinv)


CHUNKS = E_PER_TILE // CH


def _sc_agg_body(g_hbm, idxp_hbm, zeros_hbm, out_hbm,
                 acc_sh, ibuf_v, rows_v, sem_i, sem_g,
                 sem_s0, sem_s1, sem_s2, sem_s3):
    c = lax.axis_index("c")
    s = lax.axis_index("s")
    w = c * NS + s

    r0 = s * ROWS_PER_TILE
    pltpu.sync_copy(zeros_hbm.at[pl.ds(r0, ROWS_PER_TILE)],
                    acc_sh.at[pl.ds(r0, ROWS_PER_TILE)])

    crow = w * CHUNKS
    sem_s = (sem_s0, sem_s1, sem_s2, sem_s3)

    pltpu.sync_copy(idxp_hbm.at[crow], ibuf_v.at[0])
    pltpu.async_copy(idxp_hbm.at[crow + 1], ibuf_v.at[1], sem_i)
    pltpu.async_copy(idxp_hbm.at[crow + 2], ibuf_v.at[2], sem_i)
    pltpu.async_copy(idxp_hbm.at[crow + 3], ibuf_v.at[3], sem_i)
    plsc.subcore_barrier()
    pltpu.async_copy(g_hbm.at[ibuf_v.at[0, 0]], rows_v.at[0], sem_g)
    pltpu.make_async_copy(idxp_hbm.at[crow], ibuf_v.at[1], sem_i).wait()
    pltpu.async_copy(g_hbm.at[ibuf_v.at[1, 0]], rows_v.at[1], sem_g)
    pltpu.make_async_copy(idxp_hbm.at[crow], ibuf_v.at[2], sem_i).wait()
    pltpu.async_copy(g_hbm.at[ibuf_v.at[2, 0]], rows_v.at[2], sem_g)

    @pl.loop(0, CHUNKS, step=12)
    def _chunk(i0):
        for u in range(12):
            i = i0 + u
            r = u % 4
            q = u % 6
            r3 = (u + 3) % 4
            q3 = (u + 3) % 6
            q4 = (u + 4) % 6

            @pl.when(i + 3 < CHUNKS)
            def _():
                pltpu.make_async_copy(
                    idxp_hbm.at[crow], ibuf_v.at[q3], sem_i).wait()
                @pl.when(i >= 1)
                def _():
                    pltpu.make_async_copy(
                        g_hbm.at[pl.ds(0, CH)], rows_v.at[r3],
                        sem_s[r3]).wait()
                @pl.when(i + 4 < CHUNKS)
                def _():
                    pltpu.async_copy(
                        idxp_hbm.at[crow + i + 4], ibuf_v.at[q4], sem_i)
                pltpu.async_copy(
                    g_hbm.at[ibuf_v.at[q3, 0]], rows_v.at[r3], sem_g)

            pltpu.make_async_copy(
                g_hbm.at[ibuf_v.at[q, 0]], rows_v.at[r], sem_g).wait()
            pltpu.async_copy(rows_v.at[r], acc_sh.at[ibuf_v.at[q, 1]],
                             sem_s[r], add=True)

    pltpu.make_async_copy(g_hbm.at[pl.ds(0, CH)], rows_v.at[0], sem_s0).wait()
    pltpu.make_async_copy(g_hbm.at[pl.ds(0, CH)], rows_v.at[1], sem_s1).wait()
    pltpu.make_async_copy(g_hbm.at[pl.ds(0, CH)], rows_v.at[2], sem_s2).wait()
    pltpu.make_async_copy(g_hbm.at[pl.ds(0, CH)], rows_v.at[3], sem_s3).wait()

    plsc.subcore_barrier()
    pltpu.sync_copy(acc_sh.at[pl.ds(r0, ROWS_PER_TILE)],
                    out_hbm.at[pl.ds(c * N_PAD + r0, ROWS_PER_TILE)])


def _sc_agg(g_flat, idx_pair, zeros_init):
    mesh = plsc.VectorSubcoreMesh(core_axis_name="c", subcore_axis_name="s")
    fn = pl.kernel(
        _sc_agg_body,
        out_type=jax.ShapeDtypeStruct((NC * N_PAD, D), jnp.float32),
        mesh=mesh,
        scratch_types=[
            pltpu.VMEM_SHARED((N_PAD, D), jnp.float32),
            pltpu.VMEM((6, 2, CH), jnp.int32),
            pltpu.VMEM((4, CH, D), jnp.float32),
            pltpu.SemaphoreType.DMA,
            pltpu.SemaphoreType.DMA,
            pltpu.SemaphoreType.DMA,
            pltpu.SemaphoreType.DMA,
            pltpu.SemaphoreType.DMA,
            pltpu.SemaphoreType.DMA,
        ],
    )
    return fn(g_flat, idx_pair, zeros_init)


def _tc_finish_body(agg_ref, g_ref, dinv_ref, b0_ref, b1_ref,
                    wfc1_ref, bfc1_ref, wfc2_ref, bfc2_ref,
                    out_ref, acc_s):
    i = pl.program_id(0)

    @pl.when(i == 0)
    def _():
        acc_s[...] = jnp.zeros_like(acc_s)

    a0 = agg_ref[0].astype(jnp.float32) + g_ref[0].astype(jnp.float32)
    a1 = agg_ref[1].astype(jnp.float32) + g_ref[1].astype(jnp.float32)
    o0 = a0 * dinv_ref[:, 0:1] + b0_ref[...]
    o1 = a1 * dinv_ref[:, 1:2] + b1_ref[...]
    o0 = jnp.maximum(o0, 0.0)
    o1 = jnp.maximum(o1, 0.0)
    acc_s[0:1, :] += jnp.sum(o0, axis=0).reshape(1, D)
    acc_s[1:2, :] += jnp.sum(o1, axis=0).reshape(1, D)

    @pl.when(i == pl.num_programs(0) - 1)
    def _():
        hm = acc_s[0:1, :] * (1.0 / N)
        hi = acc_s[1:2, :] * (1.0 / N)
        h = jnp.dot(hm, wfc1_ref[0:D, :], preferred_element_type=jnp.float32)
        h += jnp.dot(hi, wfc1_ref[D:2 * D, :],
                     preferred_element_type=jnp.float32)
        h = jnp.maximum(h + bfc1_ref[...], 0.0)
        o = jnp.dot(h, wfc2_ref[...], preferred_element_type=jnp.float32)
        o = o + bfc2_ref[...]
        out_ref[...] = jax.nn.sigmoid(o)


def _tc_finish(agg, g, dinv, b0, b1, wfc1, bfc1, wfc2, bfc2):
    bn = 2000
    grid = (N // bn,)
    return pl.pallas_call(
        _tc_finish_body,
        grid=grid,
        in_specs=[
            pl.BlockSpec((NC, bn, D), lambda i: (0, i, 0)),
            pl.BlockSpec((NC, bn, D), lambda i: (0, i, 0)),
            pl.BlockSpec((bn, NC), lambda i: (i, 0)),
            pl.BlockSpec((1, D), lambda i: (0, 0)),
            pl.BlockSpec((1, D), lambda i: (0, 0)),
            pl.BlockSpec((2 * D, D), lambda i: (0, 0)),
            pl.BlockSpec((1, D), lambda i: (0, 0)),
            pl.BlockSpec((D, 1), lambda i: (0, 0)),
            pl.BlockSpec((1, 1), lambda i: (0, 0)),
        ],
        out_specs=pl.BlockSpec((1, 1), lambda i: (0, 0)),
        out_shape=jax.ShapeDtypeStruct((1, 1), jnp.float32),
        scratch_shapes=[pltpu.VMEM((NC, D), jnp.float32)],
    )(agg, g, dinv, b0, b1, wfc1, bfc1, wfc2, bfc2)


@jax.jit
def kernel(x_molecular, edge_index_molecular, x_interaction,
           edge_index_interaction, W_mol, b_mol, W_int, b_int,
           W_fc1, b_fc1, W_fc2, b_fc2):
    ei0 = edge_index_molecular.astype(jnp.int32)
    ei1 = edge_index_interaction.astype(jnp.int32)

    pad = jnp.zeros((E_PAD - E,), jnp.int32)
    padn = jnp.full((E_PAD - E,), N, jnp.int32)
    src_flat = jnp.concatenate([ei0[0], pad, ei1[0] + N, pad + N])
    dst_flat = jnp.concatenate([ei0[1], padn, ei1[1], padn])
    deg_dst = jnp.concatenate([ei0[1], ei1[1]])
    idx_pair = jnp.stack(
        [src_flat.reshape(-1, CH), dst_flat.reshape(-1, CH)], axis=1)

    degp = _sc_deg(deg_dst).reshape(NC, NS, N_PAD)
    dinv = _tc_dinv(degp)

    g = _tc_prescale(x_molecular, x_interaction, W_mol, W_int, dinv)
    g_flat = g.reshape(NC * N, D)

    zeros_init = jnp.zeros((N_PAD, D), jnp.float32)
    agg = _sc_agg(g_flat, idx_pair, zeros_init)

    out = _tc_finish(agg.reshape(NC, N_PAD, D), g, dinv,
                     b_mol.reshape(1, D), b_int.reshape(1, D),
                     W_fc1, b_fc1.reshape(1, D),
                     W_fc2, b_fc2.reshape(1, 1))
    return out.reshape(1)
